# speedup vs baseline: 48.1080x; 48.1080x over previous
import functools

import jax
import jax.numpy as jnp
from jax import lax
from jax.experimental import pallas as pl
from jax.experimental.pallas import tpu as pltpu
from jax.experimental.pallas import tpu_sc as plsc

_B = 16384
_NW = 32
_CH = 128
_NCH = _B // (_NW * _CH)


def _gather_body(table_hbm, idx_hbm, out_hbm, idx_v, rows_v, sem):
    wid = lax.axis_index("s") * 2 + lax.axis_index("c")
    pltpu.sync_copy(idx_hbm.at[wid], idx_v)
    for j in range(_NCH):
        pltpu.async_copy(table_hbm.at[idx_v.at[j]], rows_v.at[j], sem).wait()
    pltpu.sync_copy(rows_v, out_hbm.at[wid])


@jax.jit
def _sc_gather(table, idx):
    return pl.kernel(
        _gather_body,
        out_type=jax.ShapeDtypeStruct((_NW, _NCH, _CH), jnp.float32),
        mesh=plsc.VectorSubcoreMesh(core_axis_name="c", subcore_axis_name="s"),
        scratch_types=[
            pltpu.VMEM((_NCH, _CH), jnp.int32),
            pltpu.VMEM((_NCH, _CH), jnp.float32),
            pltpu.SemaphoreType.DMA,
        ],
    )(table, idx)


def kernel(x, kernel):
    idx = x.reshape(_NW, _NCH, _CH)
    table = kernel.reshape(-1)
    out = _sc_gather(table, idx)
    return out.reshape(_B, 1)


# speedup vs baseline: 52.0688x; 1.0823x over previous
import functools

import jax
import jax.numpy as jnp
from jax import lax
from jax.experimental import pallas as pl
from jax.experimental.pallas import tpu as pltpu
from jax.experimental.pallas import tpu_sc as plsc

_B = 16384
_NW = 32
_CH = 128
_NCH = _B // (_NW * _CH)


def _gather_body(table_hbm, idx_hbm, out_hbm, idx_v, rows_v, sem):
    wid = lax.axis_index("s") * 2 + lax.axis_index("c")
    pltpu.sync_copy(idx_hbm.at[wid], idx_v)
    copies = [
        pltpu.async_copy(table_hbm.at[idx_v.at[j]], rows_v.at[j], sem)
        for j in range(_NCH)
    ]
    for c in copies:
        c.wait()
    pltpu.sync_copy(rows_v, out_hbm.at[wid])


@jax.jit
def _sc_gather(table, idx):
    return pl.kernel(
        _gather_body,
        out_type=jax.ShapeDtypeStruct((_NW, _NCH, _CH), jnp.float32),
        mesh=plsc.VectorSubcoreMesh(core_axis_name="c", subcore_axis_name="s"),
        scratch_types=[
            pltpu.VMEM((_NCH, _CH), jnp.int32),
            pltpu.VMEM((_NCH, _CH), jnp.float32),
            pltpu.SemaphoreType.DMA,
        ],
    )(table, idx)


def kernel(x, kernel):
    idx = x.reshape(_NW, _NCH, _CH)
    table = kernel.reshape(-1)
    out = _sc_gather(table, idx)
    return out.reshape(_B, 1)


# speedup vs baseline: 53.9610x; 1.0363x over previous
import functools

import jax
import jax.numpy as jnp
from jax import lax
from jax.experimental import pallas as pl
from jax.experimental.pallas import tpu as pltpu
from jax.experimental.pallas import tpu_sc as plsc

_B = 16384
_NW = 16
_CH = 128
_NCH = _B // (_NW * _CH)


def _gather_body(table_hbm, idx_hbm, out_hbm, idx_v, rows_v, sem):
    wid = lax.axis_index("s") + lax.axis_index("c") * 16
    pltpu.sync_copy(idx_hbm.at[wid], idx_v)
    copies = [
        pltpu.async_copy(table_hbm.at[idx_v.at[j]], rows_v.at[j], sem)
        for j in range(_NCH)
    ]
    for c in copies:
        c.wait()
    pltpu.sync_copy(rows_v, out_hbm.at[wid])


@jax.jit
def _sc_gather(table, idx):
    return pl.kernel(
        _gather_body,
        out_type=jax.ShapeDtypeStruct((_NW, _NCH, _CH), jnp.float32),
        mesh=plsc.VectorSubcoreMesh(
            core_axis_name="c", subcore_axis_name="s", num_cores=1
        ),
        scratch_types=[
            pltpu.VMEM((_NCH, _CH), jnp.int32),
            pltpu.VMEM((_NCH, _CH), jnp.float32),
            pltpu.SemaphoreType.DMA,
        ],
    )(table, idx)


def kernel(x, kernel):
    idx = x.reshape(_NW, _NCH, _CH)
    table = kernel.reshape(-1)
    out = _sc_gather(table, idx)
    return out.reshape(_B, 1)


# speedup vs baseline: 54.1907x; 1.0043x over previous
import functools

import jax
import jax.numpy as jnp
from jax import lax
from jax.experimental import pallas as pl
from jax.experimental.pallas import tpu as pltpu
from jax.experimental.pallas import tpu_sc as plsc

_B = 16384
_NW = 16
_CH = 1024
_NCH = _B // (_NW * _CH)


def _gather_body(table_hbm, idx_hbm, out_hbm, idx_v, rows_v, sem):
    wid = lax.axis_index("s") + lax.axis_index("c") * 16
    pltpu.sync_copy(idx_hbm.at[wid], idx_v)
    copies = [
        pltpu.async_copy(table_hbm.at[idx_v.at[j]], rows_v.at[j], sem)
        for j in range(_NCH)
    ]
    for c in copies:
        c.wait()
    pltpu.sync_copy(rows_v, out_hbm.at[wid])


@jax.jit
def _sc_gather(table, idx):
    return pl.kernel(
        _gather_body,
        out_type=jax.ShapeDtypeStruct((_NW, _NCH, _CH), jnp.float32),
        mesh=plsc.VectorSubcoreMesh(
            core_axis_name="c", subcore_axis_name="s", num_cores=1
        ),
        scratch_types=[
            pltpu.VMEM((_NCH, _CH), jnp.int32),
            pltpu.VMEM((_NCH, _CH), jnp.float32),
            pltpu.SemaphoreType.DMA,
        ],
    )(table, idx)


def kernel(x, kernel):
    idx = x.reshape(_NW, _NCH, _CH)
    table = kernel.reshape(-1)
    out = _sc_gather(table, idx)
    return out.reshape(_B, 1)


# speedup vs baseline: 55.3413x; 1.0212x over previous
import functools

import jax
import jax.numpy as jnp
from jax import lax
from jax.experimental import pallas as pl
from jax.experimental.pallas import tpu as pltpu
from jax.experimental.pallas import tpu_sc as plsc

_B = 16384
_NW = 16
_CH = 128
_NCH = _B // (_NW * _CH)


def _gather_body(table_hbm, idx_hbm, out_hbm, idx_v, rows_v, *sems):
    wid = lax.axis_index("s") + lax.axis_index("c") * 16
    sem_i = sems[:_NCH]
    sem_g = sems[_NCH:2 * _NCH]
    sem_o = sems[2 * _NCH:]
    idx_cp = [
        pltpu.async_copy(idx_hbm.at[wid, j], idx_v.at[j], sem_i[j])
        for j in range(_NCH)
    ]
    gather_cp = []
    for j in range(_NCH):
        idx_cp[j].wait()
        gather_cp.append(
            pltpu.async_copy(table_hbm.at[idx_v.at[j]], rows_v.at[j], sem_g[j])
        )
    out_cp = []
    for j in range(_NCH):
        gather_cp[j].wait()
        out_cp.append(
            pltpu.async_copy(rows_v.at[j], out_hbm.at[wid, j], sem_o[j])
        )
    for c in out_cp:
        c.wait()


@jax.jit
def _sc_gather(table, idx):
    return pl.kernel(
        _gather_body,
        out_type=jax.ShapeDtypeStruct((_NW, _NCH, _CH), jnp.float32),
        mesh=plsc.VectorSubcoreMesh(
            core_axis_name="c", subcore_axis_name="s", num_cores=1
        ),
        scratch_types=[
            pltpu.VMEM((_NCH, _CH), jnp.int32),
            pltpu.VMEM((_NCH, _CH), jnp.float32),
        ] + [pltpu.SemaphoreType.DMA] * (3 * _NCH),
    )(table, idx)


def kernel(x, kernel):
    idx = x.reshape(_NW, _NCH, _CH)
    table = kernel.reshape(-1)
    out = _sc_gather(table, idx)
    return out.reshape(_B, 1)
